# baseline (speedup 1.0000x reference)
import jax
import jax.numpy as jnp
from jax.experimental import pallas as pl
from jax.experimental.pallas import tpu as pltpu

_NUM_CLASSES = 80
_NUM_ANCHORS = 3
_STRIDE = 8.0
_NCH = _NUM_CLASSES + 7


def _yolo_body(anch_ref, xt_ref, xb_ref, o_ref):
    for half, x_ref in enumerate((xt_ref, xb_ref)):
      for i in range(_NUM_ANCHORS):
        aw = anch_ref[i, 0]
        ah = anch_ref[i, 1]
        v = x_ref[0, i * _NCH:(i + 1) * _NCH]
        c = jax.lax.broadcasted_iota(jnp.int32, v.shape, 0)
        gx = jax.lax.broadcasted_iota(jnp.int32, v.shape, 2).astype(jnp.float32)
        gy = jax.lax.broadcasted_iota(jnp.int32, v.shape, 1).astype(jnp.float32)
        gy = gy + (32.0 * half)
        sgn = jnp.where((c == 2) | (c == 3), 1.0, -1.0)
        ca = jnp.where(c < 2, _STRIDE, jnp.where(c >= 6, 1.0, 0.0))
        cb = jnp.where(c == 2, aw, jnp.where(c == 3, ah, 0.0))
        cc = jnp.where((c == 4) | (c == 5), 1.0, 0.0)
        add = _STRIDE * jnp.where(c == 0, gx, jnp.where(c == 1, gy, 0.0))
        e = jnp.exp(v * sgn)
        sig = 1.0 / (1.0 + e)
        w = ca * sig + cb * e + cc * v + add
        o_ref[0, i, 32 * half:32 * (half + 1)] = jnp.transpose(w, (1, 2, 0))


def kernel(x, anchors):
    B, C, G, _ = x.shape
    nA, nCh = _NUM_ANCHORS, _NCH
    H = G // 2

    out = pl.pallas_call(
        _yolo_body,
        grid=(B,),
        in_specs=[
            pl.BlockSpec(memory_space=pltpu.SMEM),
            pl.BlockSpec((1, C, H, G), lambda b: (b, 0, 0, 0)),
            pl.BlockSpec((1, C, H, G), lambda b: (b, 0, 1, 0)),
        ],
        out_specs=pl.BlockSpec((1, nA, G, G, nCh), lambda b: (b, 0, 0, 0, 0)),
        out_shape=jax.ShapeDtypeStruct((B, nA, G, G, nCh), jnp.float32),
        compiler_params=pltpu.CompilerParams(
            dimension_semantics=("arbitrary",),
        ),
    )(anchors, x, x)
    return out.reshape(B, nA * G * G, nCh)


# speedup vs baseline: 1.1617x; 1.1617x over previous
import jax
import jax.numpy as jnp
from jax.experimental import pallas as pl
from jax.experimental.pallas import tpu as pltpu

_NUM_CLASSES = 80
_NUM_ANCHORS = 3
_STRIDE = 8.0
_NCH = _NUM_CLASSES + 7


def _yolo_body(anch_ref, x_ref, o_ref):
    for i in range(_NUM_ANCHORS):
        aw = anch_ref[i, 0]
        ah = anch_ref[i, 1]
        v = x_ref[0, i * _NCH:(i + 1) * _NCH]
        c = jax.lax.broadcasted_iota(jnp.int32, v.shape, 0)
        gx = jax.lax.broadcasted_iota(jnp.int32, v.shape, 2).astype(jnp.float32)
        gy = jax.lax.broadcasted_iota(jnp.int32, v.shape, 1).astype(jnp.float32)
        sgn = jnp.where((c == 2) | (c == 3), 1.0, -1.0)
        ca = jnp.where(c < 2, _STRIDE, jnp.where(c >= 6, 1.0, 0.0))
        cb = jnp.where(c == 2, aw, jnp.where(c == 3, ah, 0.0))
        cc = jnp.where((c == 4) | (c == 5), 1.0, 0.0)
        add = _STRIDE * jnp.where(c == 0, gx, jnp.where(c == 1, gy, 0.0))
        e = jnp.exp(v * sgn)
        sig = 1.0 / (1.0 + e)
        w = ca * sig + cb * e + cc * v + add
        o_ref[0, i] = jnp.transpose(w, (1, 2, 0))


def kernel(x, anchors):
    B, C, G, _ = x.shape
    nA, nCh = _NUM_ANCHORS, _NCH

    out = pl.pallas_call(
        _yolo_body,
        grid=(B,),
        in_specs=[
            pl.BlockSpec(memory_space=pltpu.SMEM),
            pl.BlockSpec((1, C, G, G), lambda b: (b, 0, 0, 0)),
        ],
        out_specs=pl.BlockSpec((1, nA, G, G, nCh), lambda b: (b, 0, 0, 0, 0)),
        out_shape=jax.ShapeDtypeStruct((B, nA, G, G, nCh), jnp.float32),
        compiler_params=pltpu.CompilerParams(
            dimension_semantics=("arbitrary",),
        ),
    )(anchors, x)
    return out.reshape(B, nA * G * G, nCh)


# speedup vs baseline: 1.2731x; 1.0959x over previous
import jax
import jax.numpy as jnp
from jax.experimental import pallas as pl
from jax.experimental.pallas import tpu as pltpu

_NUM_CLASSES = 80
_NUM_ANCHORS = 3
_STRIDE = 8.0
_NCH = _NUM_CLASSES + 7


def _yolo_body(anch_ref, x_ref, o_ref):
    for i in range(_NUM_ANCHORS):
        aw = anch_ref[i, 0]
        ah = anch_ref[i, 1]
        v = x_ref[0, i * _NCH:(i + 1) * _NCH]
        c = jax.lax.broadcasted_iota(jnp.int32, v.shape, 0)
        s = jax.lax.broadcasted_iota(jnp.int32, v.shape, 1)
        l = jax.lax.broadcasted_iota(jnp.int32, v.shape, 2)
        gx = (l % 64).astype(jnp.float32)
        gy = (s * 2 + l // 64).astype(jnp.float32)
        sgn = jnp.where((c == 2) | (c == 3), 1.0, -1.0)
        ca = jnp.where(c < 2, _STRIDE, jnp.where(c >= 6, 1.0, 0.0))
        cb = jnp.where(c == 2, aw, jnp.where(c == 3, ah, 0.0))
        cc = jnp.where((c == 4) | (c == 5), 1.0, 0.0)
        add = _STRIDE * jnp.where(c == 0, gx, jnp.where(c == 1, gy, 0.0))
        e = jnp.exp(v * sgn)
        sig = 1.0 / (1.0 + e)
        w = ca * sig + cb * e + cc * v + add
        o_ref[0, i] = jnp.transpose(w, (1, 2, 0))


def kernel(x, anchors):
    B, C, G, _ = x.shape
    nA, nCh = _NUM_ANCHORS, _NCH
    xr = x.reshape(B, C, 32, 128)

    out = pl.pallas_call(
        _yolo_body,
        grid=(B,),
        in_specs=[
            pl.BlockSpec(memory_space=pltpu.SMEM),
            pl.BlockSpec((1, C, 32, 128), lambda b: (b, 0, 0, 0)),
        ],
        out_specs=pl.BlockSpec(
            (1, nA, 32, 128, nCh), lambda b: (b, 0, 0, 0, 0)),
        out_shape=jax.ShapeDtypeStruct((B, nA, 32, 128, nCh), jnp.float32),
        compiler_params=pltpu.CompilerParams(
            dimension_semantics=("arbitrary",),
        ),
    )(anchors, xr)
    return out.reshape(B, nA * G * G, nCh)
